# baseline (speedup 1.0000x reference)
import jax
import jax.numpy as jnp
from jax.experimental import pallas as pl
from jax.experimental.pallas import tpu as pltpu

T = 10.0
SCALE = 1.0

_BN = 256


def _fused_kernel(x_ref, wA_ref, bA_ref, wB_ref, bB_ref, wD_ref, bD_ref,
                  wE_ref, bE_ref, o_ref, wAT_s, Wc_s, bc_s):
    G, Cg, Hg = wB_ref.shape
    xb = x_ref[0]

    @pl.when(pl.program_id(1) == 0)
    def _prep():
        x0 = xb[0:1, :]
        logits = (jax.lax.dot_general(
            x0, wE_ref[...], (((1,), (1,)), ((), ())),
            preferred_element_type=jnp.float32) + bE_ref[...]) / T
        w = jax.nn.softmax(logits, axis=-1)
        w0 = w[0, 0] * SCALE
        w1 = w[0, 1] * SCALE
        wAT_s[...] = wA_ref[...].astype(jnp.bfloat16).T
        for g in range(G):
            Wc_s[g] = (wB_ref[g] * w0 + wD_ref[g] * w1
                       ).astype(jnp.bfloat16).T
        bc_s[...] = bB_ref[...] * w0 + bD_ref[...] * w1

    h = jnp.dot(xb.astype(jnp.bfloat16), wAT_s[...],
                preferred_element_type=jnp.float32) + bA_ref[...]
    hb = h.astype(jnp.bfloat16)
    for g in range(G):
        o_ref[0, :, g * Cg:(g + 1) * Cg] = (
            jnp.dot(hb[:, g * Hg:(g + 1) * Hg], Wc_s[g],
                    preferred_element_type=jnp.float32)
            + bc_s[0:1, g * Cg:(g + 1) * Cg] + xb[:, g * Cg:(g + 1) * Cg])


def kernel(x, wA, bA, wB, bB, wD, bD, wE, bE):
    B, N, C = x.shape
    H = wA.shape[0]
    G, Cg, Hg = wB.shape

    grid = (B, N // _BN)
    out = pl.pallas_call(
        _fused_kernel,
        grid=grid,
        in_specs=[
            pl.BlockSpec((1, _BN, C), lambda b, n: (b, n, 0)),
            pl.BlockSpec((H, C), lambda b, n: (0, 0)),
            pl.BlockSpec((1, H), lambda b, n: (0, 0)),
            pl.BlockSpec((G, Cg, Hg), lambda b, n: (0, 0, 0)),
            pl.BlockSpec((1, C), lambda b, n: (0, 0)),
            pl.BlockSpec((G, Cg, Hg), lambda b, n: (0, 0, 0)),
            pl.BlockSpec((1, C), lambda b, n: (0, 0)),
            pl.BlockSpec((2, C), lambda b, n: (0, 0)),
            pl.BlockSpec((1, 2), lambda b, n: (0, 0)),
        ],
        out_specs=pl.BlockSpec((1, _BN, C), lambda b, n: (b, n, 0)),
        out_shape=jax.ShapeDtypeStruct((B, N, C), x.dtype),
        scratch_shapes=[
            pltpu.VMEM((C, H), jnp.bfloat16),
            pltpu.VMEM((G, Hg, Cg), jnp.bfloat16),
            pltpu.VMEM((1, C), jnp.float32),
        ],
        compiler_params=pltpu.CompilerParams(
            dimension_semantics=("parallel", "arbitrary")),
    )(x, wA, bA.reshape(1, H), wB, bB.reshape(1, C),
      wD, bD.reshape(1, C), wE, bE.reshape(1, 2))
    return out


# speedup vs baseline: 1.0886x; 1.0886x over previous
import jax
import jax.numpy as jnp
from jax.experimental import pallas as pl
from jax.experimental.pallas import tpu as pltpu

T = 10.0
SCALE = 1.0

_BN = 256


def _fused_kernel(x0_ref, x_ref, wAT_ref, bA_ref, wBt_ref, bB_ref,
                  wDt_ref, bD_ref, wET_ref, bE_ref, o_ref):
    G, Hg, Cg = wBt_ref.shape
    x0 = x0_ref[0, 0:1, :]
    logits = (jnp.dot(x0, wET_ref[...],
                      preferred_element_type=jnp.float32) + bE_ref[...]) / T
    w = jax.nn.softmax(logits, axis=-1)
    w0 = w[0, 0] * SCALE
    w1 = w[0, 1] * SCALE

    xb = x_ref[0]
    h = jnp.dot(xb.astype(jnp.bfloat16), wAT_ref[...].astype(jnp.bfloat16),
                preferred_element_type=jnp.float32) + bA_ref[...]
    hb = h.astype(jnp.bfloat16)
    for g in range(G):
        Wc = (wBt_ref[g] * w0 + wDt_ref[g] * w1).astype(jnp.bfloat16)
        bc = (bB_ref[0, g * Cg:(g + 1) * Cg] * w0
              + bD_ref[0, g * Cg:(g + 1) * Cg] * w1)
        o_ref[0, :, g * Cg:(g + 1) * Cg] = (
            jnp.dot(hb[:, g * Hg:(g + 1) * Hg], Wc,
                    preferred_element_type=jnp.float32)
            + bc + xb[:, g * Cg:(g + 1) * Cg])


def kernel(x, wA, bA, wB, bB, wD, bD, wE, bE):
    B, N, C = x.shape
    H = wA.shape[0]
    G, Cg, Hg = wB.shape

    wAT = wA.T
    wET = wE.T
    wBt = jnp.transpose(wB, (0, 2, 1))
    wDt = jnp.transpose(wD, (0, 2, 1))

    grid = (B, N // _BN)
    out = pl.pallas_call(
        _fused_kernel,
        grid=grid,
        in_specs=[
            pl.BlockSpec((1, 8, C), lambda b, n: (b, 0, 0)),
            pl.BlockSpec((1, _BN, C), lambda b, n: (b, n, 0)),
            pl.BlockSpec((C, H), lambda b, n: (0, 0)),
            pl.BlockSpec((1, H), lambda b, n: (0, 0)),
            pl.BlockSpec((G, Hg, Cg), lambda b, n: (0, 0, 0)),
            pl.BlockSpec((1, C), lambda b, n: (0, 0)),
            pl.BlockSpec((G, Hg, Cg), lambda b, n: (0, 0, 0)),
            pl.BlockSpec((1, C), lambda b, n: (0, 0)),
            pl.BlockSpec((C, 2), lambda b, n: (0, 0)),
            pl.BlockSpec((1, 2), lambda b, n: (0, 0)),
        ],
        out_specs=pl.BlockSpec((1, _BN, C), lambda b, n: (b, n, 0)),
        out_shape=jax.ShapeDtypeStruct((B, N, C), x.dtype),
        compiler_params=pltpu.CompilerParams(
            dimension_semantics=("parallel", "arbitrary")),
    )(x, x, wAT, bA.reshape(1, H), wBt, bB.reshape(1, C),
      wDt, bD.reshape(1, C), wET, bE.reshape(1, 2))
    return out


# speedup vs baseline: 1.1419x; 1.0489x over previous
import jax
import jax.numpy as jnp
from jax.experimental import pallas as pl
from jax.experimental.pallas import tpu as pltpu

T = 10.0
SCALE = 1.0

_BN = 256


def _fused_kernel(x0_ref, x_ref, wAT_ref, bA_ref, wBt_ref, bB_ref,
                  wDt_ref, bD_ref, wE_ref, bE_ref, o_ref):
    G, Hg, Cg = wBt_ref.shape
    x0 = x0_ref[0, 0:1, :]
    l0 = jnp.sum(x0[0] * wE_ref[0]) + bE_ref[0, 0]
    l1 = jnp.sum(x0[0] * wE_ref[1]) + bE_ref[0, 1]
    w0 = jax.nn.sigmoid((l0 - l1) / T) * SCALE
    w1 = SCALE - w0

    xb = x_ref[0]
    h = jnp.dot(xb.astype(jnp.bfloat16), wAT_ref[...].astype(jnp.bfloat16),
                preferred_element_type=jnp.float32) + bA_ref[...]
    hb = h.astype(jnp.bfloat16)
    for g in range(G):
        Wc = (wBt_ref[g] * w0 + wDt_ref[g] * w1).astype(jnp.bfloat16)
        bc = (bB_ref[0, g * Cg:(g + 1) * Cg] * w0
              + bD_ref[0, g * Cg:(g + 1) * Cg] * w1)
        o_ref[0, :, g * Cg:(g + 1) * Cg] = (
            jnp.dot(hb[:, g * Hg:(g + 1) * Hg], Wc,
                    preferred_element_type=jnp.float32)
            + bc + xb[:, g * Cg:(g + 1) * Cg])


def kernel(x, wA, bA, wB, bB, wD, bD, wE, bE):
    B, N, C = x.shape
    H = wA.shape[0]
    G, Cg, Hg = wB.shape

    wAT = wA.T
    wBt = jnp.transpose(wB, (0, 2, 1))
    wDt = jnp.transpose(wD, (0, 2, 1))

    grid = (B, N // _BN)
    out = pl.pallas_call(
        _fused_kernel,
        grid=grid,
        in_specs=[
            pl.BlockSpec((1, 8, C), lambda b, n: (b, 0, 0)),
            pl.BlockSpec((1, _BN, C), lambda b, n: (b, n, 0)),
            pl.BlockSpec((C, H), lambda b, n: (0, 0)),
            pl.BlockSpec((1, H), lambda b, n: (0, 0)),
            pl.BlockSpec((G, Hg, Cg), lambda b, n: (0, 0, 0)),
            pl.BlockSpec((1, C), lambda b, n: (0, 0)),
            pl.BlockSpec((G, Hg, Cg), lambda b, n: (0, 0, 0)),
            pl.BlockSpec((1, C), lambda b, n: (0, 0)),
            pl.BlockSpec((2, C), lambda b, n: (0, 0)),
            pl.BlockSpec((1, 2), lambda b, n: (0, 0)),
        ],
        out_specs=pl.BlockSpec((1, _BN, C), lambda b, n: (b, n, 0)),
        out_shape=jax.ShapeDtypeStruct((B, N, C), x.dtype),
        compiler_params=pltpu.CompilerParams(
            dimension_semantics=("parallel", "arbitrary")),
    )(x, x, wAT, bA.reshape(1, H), wBt, bB.reshape(1, C),
      wDt, bD.reshape(1, C), wE, bE.reshape(1, 2))
    return out
